# Initial kernel scaffold; baseline (speedup 1.0000x reference)
import functools

import jax
import jax.numpy as jnp
from jax import lax
from jax.experimental import pallas as pl
from jax.experimental.pallas import tpu as pltpu
from jax.experimental.pallas import tpu_sc as plsc

_NC = 2
_NS = 16
_L = 16
_C = 80
_BLK = 1000


def _sc_deg_body(col_hbm, ew_hbm, out_hbm, col_v, ew_v, zb_v, acc_sh):
    cid = lax.axis_index("c")
    sid = lax.axis_index("s")
    wid = cid * _NS + sid
    e_total = col_hbm.shape[0]
    n = acc_sh.shape[0]
    epw = e_total // (_NC * _NS)
    base = wid * epw

    def _zb_zero(i, _):
        zb_v[pl.ds(i * _L, _L)] = jnp.zeros((_L,), jnp.float32)
        return 0
    lax.fori_loop(0, zb_v.shape[0] // _L, _zb_zero, 0)

    @pl.when(sid == 0)
    def _():
        pltpu.sync_copy(zb_v.at[pl.ds(0, n)], acc_sh)
    plsc.subcore_barrier()

    def _chunk(i, _):
        off = base + i * _C
        pltpu.sync_copy(col_hbm.at[pl.ds(off, _C)], col_v)
        pltpu.sync_copy(ew_hbm.at[pl.ds(off, _C)], ew_v)
        pltpu.sync_copy(ew_v, acc_sh.at[col_v], add=True)
        return 0
    lax.fori_loop(0, epw // _C, _chunk, 0)
    plsc.subcore_barrier()

    @pl.when(sid == 0)
    def _():
        pltpu.sync_copy(acc_sh, out_hbm.at[cid])


def _sc_deg(col, ew, n):
    mesh = plsc.VectorSubcoreMesh(core_axis_name="c", subcore_axis_name="s")
    f = pl.kernel(
        _sc_deg_body,
        out_type=jax.ShapeDtypeStruct((_NC, n), jnp.float32),
        mesh=mesh,
        scratch_types=[
            pltpu.VMEM((_C,), jnp.int32),
            pltpu.VMEM((_C,), jnp.float32),
            pltpu.VMEM((n,), jnp.float32),
            pltpu.VMEM_SHARED((n,), jnp.float32),
        ],
    )
    return f(col, ew)


def _sc_agg_body(row_hbm, col_hbm, ew_hbm, y_hbm, out_hbm,
                 row_v, col_v, ew_v, rows_v, zb_v, acc_sh, sem):
    cid = lax.axis_index("c")
    sid = lax.axis_index("s")
    wid = cid * _NS + sid
    e_total = row_hbm.shape[0]
    n, fdim = y_hbm.shape
    nf = fdim // _L
    epw = e_total // (_NC * _NS)
    base = wid * epw
    rpt = n // _NS
    zc = 125

    def _zb_zero(i, _):
        for f in range(nf):
            zb_v[i, pl.ds(f * _L, _L)] = jnp.zeros((_L,), jnp.float32)
        return 0
    lax.fori_loop(0, zb_v.shape[0], _zb_zero, 0)
    for k in range(rpt // zc):
        pltpu.sync_copy(zb_v.at[pl.ds(0, zc)],
                        acc_sh.at[pl.ds(sid * rpt + k * zc, zc)])
    plsc.subcore_barrier()

    def _chunk(i, _):
        off = base + i * _C
        pltpu.sync_copy(row_hbm.at[pl.ds(off, _C)], row_v)
        pltpu.sync_copy(col_hbm.at[pl.ds(off, _C)], col_v)
        pltpu.sync_copy(ew_hbm.at[pl.ds(off, _C)], ew_v)
        pltpu.async_copy(y_hbm.at[row_v], rows_v, sem).wait()

        def _scale(e, _):
            s = plsc.load_gather(ew_v, [jnp.full((_L,), e, jnp.int32)])
            for f in range(nf):
                rows_v[e, pl.ds(f * _L, _L)] = rows_v[e, pl.ds(f * _L, _L)] * s
            return 0
        lax.fori_loop(0, _C, _scale, 0)
        pltpu.sync_copy(rows_v, acc_sh.at[col_v], add=True)
        return 0
    lax.fori_loop(0, epw // _C, _chunk, 0)
    plsc.subcore_barrier()

    pltpu.sync_copy(acc_sh.at[pl.ds(sid * rpt, rpt)],
                    out_hbm.at[cid, pl.ds(sid * rpt, rpt)])


def _sc_agg(row, col, ew, y):
    n, fdim = y.shape
    mesh = plsc.VectorSubcoreMesh(core_axis_name="c", subcore_axis_name="s")
    f = pl.kernel(
        _sc_agg_body,
        out_type=jax.ShapeDtypeStruct((_NC, n, fdim), jnp.float32),
        mesh=mesh,
        scratch_types=[
            pltpu.VMEM((_C,), jnp.int32),
            pltpu.VMEM((_C,), jnp.int32),
            pltpu.VMEM((_C,), jnp.float32),
            pltpu.VMEM((_C, fdim), jnp.float32),
            pltpu.VMEM((128, fdim), jnp.float32),
            pltpu.VMEM_SHARED((n, fdim), jnp.float32),
            pltpu.SemaphoreType.DMA,
        ],
    )
    return f(row, col, ew, y)


def _dinv_of(dp):
    return lax.rsqrt(1.0 + dp[0] + dp[1])


def _tc_mm1_body(dp_ref, x_ref, w_ref, y_ref):
    dinv = _dinv_of(dp_ref[...])
    y_ref[...] = dinv * jnp.dot(x_ref[...], w_ref[...],
                                preferred_element_type=jnp.float32)


def _tc_mm1(dp3, x, w):
    n, fdim = x.shape
    grid = (n // _BLK,)
    return pl.pallas_call(
        _tc_mm1_body,
        grid=grid,
        in_specs=[
            pl.BlockSpec((_NC, _BLK, 1), lambda i: (0, i, 0)),
            pl.BlockSpec((_BLK, fdim), lambda i: (i, 0)),
            pl.BlockSpec((fdim, w.shape[1]), lambda i: (0, 0)),
        ],
        out_specs=pl.BlockSpec((_BLK, w.shape[1]), lambda i: (i, 0)),
        out_shape=jax.ShapeDtypeStruct((n, w.shape[1]), jnp.float32),
    )(dp3, x, w)


def _tc_post_body(dp_ref, p_ref, y_ref, b_ref, v_ref, st_ref):
    i = pl.program_id(0)
    dinv = _dinv_of(dp_ref[...])
    p = p_ref[...]
    v = dinv * (p[0] + p[1] + y_ref[...]) + b_ref[...][None, :]
    v_ref[...] = v

    @pl.when(i == 0)
    def _():
        st_ref[...] = jnp.zeros_like(st_ref)
    st_ref[0:1, :] += jnp.sum(v, axis=0, keepdims=True)
    st_ref[1:2, :] += jnp.sum(v * v, axis=0, keepdims=True)


def _tc_post(dp3, parts, y, b):
    n, fdim = y.shape
    grid = (n // _BLK,)
    return pl.pallas_call(
        _tc_post_body,
        grid=grid,
        in_specs=[
            pl.BlockSpec((_NC, _BLK, 1), lambda i: (0, i, 0)),
            pl.BlockSpec((_NC, _BLK, fdim), lambda i: (0, i, 0)),
            pl.BlockSpec((_BLK, fdim), lambda i: (i, 0)),
            pl.BlockSpec((fdim,), lambda i: (0,)),
        ],
        out_specs=[
            pl.BlockSpec((_BLK, fdim), lambda i: (i, 0)),
            pl.BlockSpec((8, fdim), lambda i: (0, 0)),
        ],
        out_shape=[
            jax.ShapeDtypeStruct((n, fdim), jnp.float32),
            jax.ShapeDtypeStruct((8, fdim), jnp.float32),
        ],
    )(dp3, parts, y, b)


def _bn_apply(v, st, g_row, b_row, n_nodes):
    mean = st[0:1, :] * (1.0 / n_nodes)
    var = st[1:2, :] * (1.0 / n_nodes) - mean * mean
    return g_row[None, :] * ((v - mean) * lax.rsqrt(var + 1e-5)) + b_row[None, :]


def _tc_mm2_body(dp_ref, v_ref, st_ref, g_ref, be_ref, w_ref, y2_ref, *, n_nodes):
    z = _bn_apply(v_ref[...], st_ref[...], g_ref[...], be_ref[...], n_nodes)
    h = jnp.where(z >= 0, z, 0.01 * z)
    dinv = _dinv_of(dp_ref[...])
    y2_ref[...] = dinv * jnp.dot(h, w_ref[...],
                                 preferred_element_type=jnp.float32)


def _tc_mm2(dp3, v, st, g, be, w):
    n, fdim = v.shape
    grid = (n // _BLK,)
    return pl.pallas_call(
        functools.partial(_tc_mm2_body, n_nodes=float(n)),
        grid=grid,
        in_specs=[
            pl.BlockSpec((_NC, _BLK, 1), lambda i: (0, i, 0)),
            pl.BlockSpec((_BLK, fdim), lambda i: (i, 0)),
            pl.BlockSpec((8, fdim), lambda i: (0, 0)),
            pl.BlockSpec((fdim,), lambda i: (0,)),
            pl.BlockSpec((fdim,), lambda i: (0,)),
            pl.BlockSpec((fdim, w.shape[1]), lambda i: (0, 0)),
        ],
        out_specs=pl.BlockSpec((_BLK, w.shape[1]), lambda i: (i, 0)),
        out_shape=jax.ShapeDtypeStruct((n, w.shape[1]), jnp.float32),
    )(dp3, v, st, g, be, w)


def _tc_final_body(v_ref, st_ref, g_ref, be_ref, o_ref, *, n_nodes):
    o_ref[...] = _bn_apply(v_ref[...], st_ref[...], g_ref[...], be_ref[...],
                           n_nodes)


def _tc_final(v, st, g, be):
    n, fdim = v.shape
    grid = (n // _BLK,)
    return pl.pallas_call(
        functools.partial(_tc_final_body, n_nodes=float(n)),
        grid=grid,
        in_specs=[
            pl.BlockSpec((_BLK, fdim), lambda i: (i, 0)),
            pl.BlockSpec((8, fdim), lambda i: (0, 0)),
            pl.BlockSpec((fdim,), lambda i: (0,)),
            pl.BlockSpec((fdim,), lambda i: (0,)),
        ],
        out_specs=pl.BlockSpec((_BLK, fdim), lambda i: (i, 0)),
        out_shape=jax.ShapeDtypeStruct((n, fdim), jnp.float32),
    )(v, st, g, be)


def kernel(ATC_adj, ATC_weight, drug_smiles_fea, W1, b1, gamma1, beta1,
           W2, b2, gamma2, beta2):
    x = drug_smiles_fea
    n = x.shape[0]
    row = ATC_adj[0].astype(jnp.int32)
    col = ATC_adj[1].astype(jnp.int32)
    ew = ATC_weight.astype(jnp.float32)

    dparts = _sc_deg(col, ew, n)
    dp3 = dparts.reshape(_NC, n, 1)

    y1 = _tc_mm1(dp3, x, W1)
    parts1 = _sc_agg(row, col, ew, y1)
    v1, st1 = _tc_post(dp3, parts1, y1, b1)

    y2 = _tc_mm2(dp3, v1, st1, gamma1, beta1, W2)
    parts2 = _sc_agg(row, col, ew, y2)
    v2, st2 = _tc_post(dp3, parts2, y2, b2)

    return _tc_final(v2, st2, gamma2, beta2)


# speedup vs baseline: 9.8740x; 9.8740x over previous
import functools

import jax
import jax.numpy as jnp
from jax import lax
from jax.experimental import pallas as pl
from jax.experimental.pallas import tpu as pltpu
from jax.experimental.pallas import tpu_sc as plsc

_NC = 2
_NS = 16
_L = 16
_C = 80
_BLK = 1000


def _sc_deg_body(col_hbm, ew_hbm, out_hbm, col_v, ew_v, zb_v, acc_sh):
    cid = lax.axis_index("c")
    sid = lax.axis_index("s")
    wid = cid * _NS + sid
    e_total = col_hbm.shape[0]
    n = acc_sh.shape[0]
    epw = e_total // (_NC * _NS)
    base = wid * epw

    def _zb_zero(i, _):
        zb_v[pl.ds(i * _L, _L)] = jnp.zeros((_L,), jnp.float32)
        return 0
    lax.fori_loop(0, zb_v.shape[0] // _L, _zb_zero, 0)

    @pl.when(sid == 0)
    def _():
        pltpu.sync_copy(zb_v.at[pl.ds(0, n)], acc_sh)
    plsc.subcore_barrier()

    def _chunk(i, _):
        off = base + i * _C
        pltpu.sync_copy(col_hbm.at[pl.ds(off, _C)], col_v)
        pltpu.sync_copy(ew_hbm.at[pl.ds(off, _C)], ew_v)
        pltpu.sync_copy(ew_v, acc_sh.at[col_v], add=True)
        return 0
    lax.fori_loop(0, epw // _C, _chunk, 0)
    plsc.subcore_barrier()

    @pl.when(sid == 0)
    def _():
        pltpu.sync_copy(acc_sh, out_hbm.at[cid])


def _sc_deg(col, ew, n):
    mesh = plsc.VectorSubcoreMesh(core_axis_name="c", subcore_axis_name="s")
    f = pl.kernel(
        _sc_deg_body,
        out_type=jax.ShapeDtypeStruct((_NC, n), jnp.float32),
        mesh=mesh,
        scratch_types=[
            pltpu.VMEM((_C,), jnp.int32),
            pltpu.VMEM((_C,), jnp.float32),
            pltpu.VMEM((n,), jnp.float32),
            pltpu.VMEM_SHARED((n,), jnp.float32),
        ],
    )
    return f(col, ew)


def _sc_agg_body(row_hbm, col_hbm, ew_hbm, y_hbm, out_hbm,
                 row_v, col_v, ew_v, rows_v, zb_v, acc_sh, sem):
    cid = lax.axis_index("c")
    sid = lax.axis_index("s")
    wid = cid * _NS + sid
    e_total = row_hbm.shape[0]
    n, fdim = y_hbm.shape
    nf = fdim // _L
    epw = e_total // (_NC * _NS)
    base = wid * epw
    nwt = 10
    rpt = n // nwt
    zc = zb_v.shape[0]

    def _zb_zero(i, _):
        for f in range(nf):
            zb_v[i, pl.ds(f * _L, _L)] = jnp.zeros((_L,), jnp.float32)
        return 0
    lax.fori_loop(0, zc, _zb_zero, 0)

    @pl.when(sid < nwt)
    def _():
        for k in range(rpt // zc):
            pltpu.sync_copy(zb_v, acc_sh.at[pl.ds(sid * rpt + k * zc, zc)])
    plsc.subcore_barrier()

    def _chunk(i, _):
        off = base + i * _C
        pltpu.sync_copy(row_hbm.at[pl.ds(off, _C)], row_v)
        pltpu.sync_copy(col_hbm.at[pl.ds(off, _C)], col_v)
        pltpu.sync_copy(ew_hbm.at[pl.ds(off, _C)], ew_v)
        pltpu.async_copy(y_hbm.at[row_v], rows_v, sem).wait()

        def _scale(g, _):
            ewv = ew_v[pl.ds(g * _L, _L)]
            for j in range(_L):
                s = lax.gather(
                    ewv, jnp.full((_L, 1), j, jnp.int32),
                    dimension_numbers=lax.GatherDimensionNumbers(
                        offset_dims=(), collapsed_slice_dims=(0,),
                        start_index_map=(0,)),
                    slice_sizes=(1,),
                    mode=lax.GatherScatterMode.PROMISE_IN_BOUNDS)
                e = g * _L + j
                for f in range(nf):
                    rows_v[e, pl.ds(f * _L, _L)] = (
                        rows_v[e, pl.ds(f * _L, _L)] * s)
            return 0
        lax.fori_loop(0, _C // _L, _scale, 0)
        pltpu.sync_copy(rows_v, acc_sh.at[col_v], add=True)
        return 0
    lax.fori_loop(0, epw // _C, _chunk, 0)
    plsc.subcore_barrier()

    @pl.when(sid < nwt)
    def _():
        pltpu.sync_copy(acc_sh.at[pl.ds(sid * rpt, rpt)],
                        out_hbm.at[cid, pl.ds(sid * rpt, rpt)])


def _sc_agg(row, col, ew, y):
    n, fdim = y.shape
    mesh = plsc.VectorSubcoreMesh(core_axis_name="c", subcore_axis_name="s")
    f = pl.kernel(
        _sc_agg_body,
        out_type=jax.ShapeDtypeStruct((_NC, n, fdim), jnp.float32),
        mesh=mesh,
        scratch_types=[
            pltpu.VMEM((_C,), jnp.int32),
            pltpu.VMEM((_C,), jnp.int32),
            pltpu.VMEM((_C,), jnp.float32),
            pltpu.VMEM((_C, fdim), jnp.float32),
            pltpu.VMEM((200, fdim), jnp.float32),
            pltpu.VMEM_SHARED((n, fdim), jnp.float32),
            pltpu.SemaphoreType.DMA,
        ],
    )
    return f(row, col, ew, y)


def _dinv_of(dp):
    return lax.rsqrt(1.0 + dp[0] + dp[1])


def _tc_mm1_body(dp_ref, x_ref, w_ref, y_ref):
    dinv = _dinv_of(dp_ref[...])
    y_ref[...] = dinv * jnp.dot(x_ref[...], w_ref[...],
                                preferred_element_type=jnp.float32)


def _tc_mm1(dp3, x, w):
    n, fdim = x.shape
    grid = (n // _BLK,)
    return pl.pallas_call(
        _tc_mm1_body,
        grid=grid,
        in_specs=[
            pl.BlockSpec((_NC, _BLK, 1), lambda i: (0, i, 0)),
            pl.BlockSpec((_BLK, fdim), lambda i: (i, 0)),
            pl.BlockSpec((fdim, w.shape[1]), lambda i: (0, 0)),
        ],
        out_specs=pl.BlockSpec((_BLK, w.shape[1]), lambda i: (i, 0)),
        out_shape=jax.ShapeDtypeStruct((n, w.shape[1]), jnp.float32),
    )(dp3, x, w)


def _tc_post_body(dp_ref, p_ref, y_ref, b_ref, v_ref, st_ref):
    i = pl.program_id(0)
    dinv = _dinv_of(dp_ref[...])
    p = p_ref[...]
    v = dinv * (p[0] + p[1] + y_ref[...]) + b_ref[...][None, :]
    v_ref[...] = v

    @pl.when(i == 0)
    def _():
        st_ref[...] = jnp.zeros_like(st_ref)
    st_ref[0:1, :] += jnp.sum(v, axis=0, keepdims=True)
    st_ref[1:2, :] += jnp.sum(v * v, axis=0, keepdims=True)


def _tc_post(dp3, parts, y, b):
    n, fdim = y.shape
    grid = (n // _BLK,)
    return pl.pallas_call(
        _tc_post_body,
        grid=grid,
        in_specs=[
            pl.BlockSpec((_NC, _BLK, 1), lambda i: (0, i, 0)),
            pl.BlockSpec((_NC, _BLK, fdim), lambda i: (0, i, 0)),
            pl.BlockSpec((_BLK, fdim), lambda i: (i, 0)),
            pl.BlockSpec((fdim,), lambda i: (0,)),
        ],
        out_specs=[
            pl.BlockSpec((_BLK, fdim), lambda i: (i, 0)),
            pl.BlockSpec((8, fdim), lambda i: (0, 0)),
        ],
        out_shape=[
            jax.ShapeDtypeStruct((n, fdim), jnp.float32),
            jax.ShapeDtypeStruct((8, fdim), jnp.float32),
        ],
    )(dp3, parts, y, b)


def _bn_apply(v, st, g_row, b_row, n_nodes):
    mean = st[0:1, :] * (1.0 / n_nodes)
    var = st[1:2, :] * (1.0 / n_nodes) - mean * mean
    return g_row[None, :] * ((v - mean) * lax.rsqrt(var + 1e-5)) + b_row[None, :]


def _tc_mm2_body(dp_ref, v_ref, st_ref, g_ref, be_ref, w_ref, y2_ref, *, n_nodes):
    z = _bn_apply(v_ref[...], st_ref[...], g_ref[...], be_ref[...], n_nodes)
    h = jnp.where(z >= 0, z, 0.01 * z)
    dinv = _dinv_of(dp_ref[...])
    y2_ref[...] = dinv * jnp.dot(h, w_ref[...],
                                 preferred_element_type=jnp.float32)


def _tc_mm2(dp3, v, st, g, be, w):
    n, fdim = v.shape
    grid = (n // _BLK,)
    return pl.pallas_call(
        functools.partial(_tc_mm2_body, n_nodes=float(n)),
        grid=grid,
        in_specs=[
            pl.BlockSpec((_NC, _BLK, 1), lambda i: (0, i, 0)),
            pl.BlockSpec((_BLK, fdim), lambda i: (i, 0)),
            pl.BlockSpec((8, fdim), lambda i: (0, 0)),
            pl.BlockSpec((fdim,), lambda i: (0,)),
            pl.BlockSpec((fdim,), lambda i: (0,)),
            pl.BlockSpec((fdim, w.shape[1]), lambda i: (0, 0)),
        ],
        out_specs=pl.BlockSpec((_BLK, w.shape[1]), lambda i: (i, 0)),
        out_shape=jax.ShapeDtypeStruct((n, w.shape[1]), jnp.float32),
    )(dp3, v, st, g, be, w)


def _tc_final_body(v_ref, st_ref, g_ref, be_ref, o_ref, *, n_nodes):
    o_ref[...] = _bn_apply(v_ref[...], st_ref[...], g_ref[...], be_ref[...],
                           n_nodes)


def _tc_final(v, st, g, be):
    n, fdim = v.shape
    grid = (n // _BLK,)
    return pl.pallas_call(
        functools.partial(_tc_final_body, n_nodes=float(n)),
        grid=grid,
        in_specs=[
            pl.BlockSpec((_BLK, fdim), lambda i: (i, 0)),
            pl.BlockSpec((8, fdim), lambda i: (0, 0)),
            pl.BlockSpec((fdim,), lambda i: (0,)),
            pl.BlockSpec((fdim,), lambda i: (0,)),
        ],
        out_specs=pl.BlockSpec((_BLK, fdim), lambda i: (i, 0)),
        out_shape=jax.ShapeDtypeStruct((n, fdim), jnp.float32),
    )(v, st, g, be)


def kernel(ATC_adj, ATC_weight, drug_smiles_fea, W1, b1, gamma1, beta1,
           W2, b2, gamma2, beta2):
    x = drug_smiles_fea
    n = x.shape[0]
    row = ATC_adj[0].astype(jnp.int32)
    col = ATC_adj[1].astype(jnp.int32)
    ew = ATC_weight.astype(jnp.float32)

    dparts = _sc_deg(col, ew, n)
    dp3 = dparts.reshape(_NC, n, 1)

    y1 = _tc_mm1(dp3, x, W1)
    parts1 = _sc_agg(row, col, ew, y1)
    v1, st1 = _tc_post(dp3, parts1, y1, b1)

    y2 = _tc_mm2(dp3, v1, st1, gamma1, beta1, W2)
    parts2 = _sc_agg(row, col, ew, y2)
    v2, st2 = _tc_post(dp3, parts2, y2, b2)

    return _tc_final(v2, st2, gamma2, beta2)


# speedup vs baseline: 10.1294x; 1.0259x over previous
import functools

import jax
import jax.numpy as jnp
from jax import lax
from jax.experimental import pallas as pl
from jax.experimental.pallas import tpu as pltpu
from jax.experimental.pallas import tpu_sc as plsc

_NC = 2
_NS = 16
_L = 16
_C = 80
_BLK = 1000


def _sc_deg_body(col_hbm, ew_hbm, z_hbm, out_hbm, col_v, ew_v, acc_sh, sem):
    cid = lax.axis_index("c")
    sid = lax.axis_index("s")
    wid = cid * _NS + sid
    nch = col_v.shape[0]
    n = acc_sh.shape[0]

    pltpu.sync_copy(col_hbm.at[wid], col_v)
    pltpu.sync_copy(ew_hbm.at[wid], ew_v)

    @pl.when(sid == 0)
    def _():
        pltpu.sync_copy(z_hbm, acc_sh)
    plsc.subcore_barrier()

    def _fire(i, _):
        pltpu.async_copy(ew_v.at[i], acc_sh.at[col_v.at[i]], sem, add=True)
        return 0
    lax.fori_loop(0, nch, _fire, 0)

    def _drain(i, _):
        pltpu.make_async_copy(ew_v.at[0], acc_sh.at[col_v.at[0]], sem).wait()
        return 0
    lax.fori_loop(0, nch, _drain, 0)
    plsc.subcore_barrier()

    @pl.when(sid == 0)
    def _():
        pltpu.sync_copy(acc_sh, out_hbm.at[cid])


def _sc_deg(col3, ew3, zeros1, n):
    nch, c = col3.shape[1], col3.shape[2]
    mesh = plsc.VectorSubcoreMesh(core_axis_name="c", subcore_axis_name="s",
                                  num_cores=_NC, num_subcores=_NS)
    f = pl.kernel(
        _sc_deg_body,
        out_type=jax.ShapeDtypeStruct((_NC, n), jnp.float32),
        mesh=mesh,
        scratch_types=[
            pltpu.VMEM((nch, c), jnp.int32),
            pltpu.VMEM((nch, c), jnp.float32),
            pltpu.VMEM_SHARED((n,), jnp.float32),
            pltpu.SemaphoreType.DMA,
        ],
    )
    return f(col3, ew3, zeros1)


_BCH = 8


def _sc_agg_body(row_hbm, col_hbm, ew_hbm, y_hbm, z_hbm, out_hbm,
                 row0, row1, ew0, ew1, c0, c1, rows0, rows1,
                 acc_sh, gsem, ssem, isem, *, nch, c):
    cid = lax.axis_index("c")
    sid = lax.axis_index("s")
    wid = cid * _NS + sid
    n, fdim = y_hbm.shape
    nf = fdim // _L
    nwt = 10
    rpt = n // nwt
    rowv = (row0, row1)
    ewv = (ew0, ew1)
    colv = (c0, c1)
    rows = (rows0, rows1)
    base = wid * (nch * c)

    def _load_idx(i, b):
        off = base + i * c
        pltpu.sync_copy(row_hbm.at[pl.ds(off, c)], rowv[b])
        pltpu.sync_copy(ew_hbm.at[pl.ds(off, c)], ewv[b])
        pltpu.sync_copy(col_hbm.at[pl.ds(off, c)], colv[b])

    @pl.when(sid < nwt)
    def _():
        pltpu.sync_copy(z_hbm.at[pl.ds(sid * rpt, rpt)],
                        acc_sh.at[pl.ds(sid * rpt, rpt)])
    plsc.subcore_barrier()

    def _outer(i, _):
        b = 0
        _load_idx(i, b)
        pltpu.async_copy(y_hbm.at[rowv[b]], rows[b], gsem).wait()

        rb = rows[b]
        ewb = ewv[b]
        if True:

            def _scale(k, _):
                ewl = ewb[pl.ds(k * _L, _L)]
                for j in range(_L):
                    s = lax.gather(
                        ewl, jnp.full((_L, 1), j, jnp.int32),
                        dimension_numbers=lax.GatherDimensionNumbers(
                            offset_dims=(), collapsed_slice_dims=(0,),
                            start_index_map=(0,)),
                        slice_sizes=(1,),
                        mode=lax.GatherScatterMode.PROMISE_IN_BOUNDS)
                    e = k * _L + j
                    for f in range(nf):
                        rb[e, pl.ds(f * _L, _L)] = (
                            rb[e, pl.ds(f * _L, _L)] * s)
                return 0
            lax.fori_loop(0, c // _L, _scale, 0)

            pltpu.async_copy(rows[b], acc_sh.at[colv[b]], ssem,
                             add=True).wait()
        return 0
    lax.fori_loop(0, nch, _outer, 0)
    plsc.subcore_barrier()

    @pl.when(sid < nwt)
    def _():
        pltpu.sync_copy(acc_sh.at[pl.ds(sid * rpt, rpt)],
                        out_hbm.at[cid, pl.ds(sid * rpt, rpt)])


def _sc_agg(row3, col3, ew3, y, zeros2):
    n, fdim = y.shape
    c = row3.shape[2]
    row1 = row3.reshape(-1)
    col1 = col3.reshape(-1)
    ew1 = ew3.reshape(-1)
    mesh = plsc.VectorSubcoreMesh(core_axis_name="c", subcore_axis_name="s",
                                  num_cores=_NC, num_subcores=_NS)
    f = pl.kernel(
        functools.partial(_sc_agg_body, nch=row3.shape[1], c=c),
        out_type=jax.ShapeDtypeStruct((_NC, n, fdim), jnp.float32),
        mesh=mesh,
        scratch_types=[
            pltpu.VMEM((c,), jnp.int32),
            pltpu.VMEM((c,), jnp.int32),
            pltpu.VMEM((c,), jnp.float32),
            pltpu.VMEM((c,), jnp.float32),
            pltpu.VMEM((c,), jnp.int32),
            pltpu.VMEM((c,), jnp.int32),
            pltpu.VMEM((c, fdim), jnp.float32),
            pltpu.VMEM((c, fdim), jnp.float32),
            pltpu.VMEM_SHARED((n, fdim), jnp.float32),
            pltpu.SemaphoreType.DMA,
            pltpu.SemaphoreType.DMA,
            pltpu.SemaphoreType.DMA,
        ],
    )
    return f(row1, col1, ew1, y, zeros2)


def _dinv_of(dp):
    return lax.rsqrt(1.0 + dp[0] + dp[1])


def _tc_mm1_body(dp_ref, x_ref, w_ref, y_ref):
    dinv = _dinv_of(dp_ref[...])
    y_ref[...] = dinv * jnp.dot(x_ref[...], w_ref[...],
                                preferred_element_type=jnp.float32)


def _tc_mm1(dp3, x, w):
    n, fdim = x.shape
    grid = (n // _BLK,)
    return pl.pallas_call(
        _tc_mm1_body,
        grid=grid,
        in_specs=[
            pl.BlockSpec((_NC, _BLK, 1), lambda i: (0, i, 0)),
            pl.BlockSpec((_BLK, fdim), lambda i: (i, 0)),
            pl.BlockSpec((fdim, w.shape[1]), lambda i: (0, 0)),
        ],
        out_specs=pl.BlockSpec((_BLK, w.shape[1]), lambda i: (i, 0)),
        out_shape=jax.ShapeDtypeStruct((n, w.shape[1]), jnp.float32),
    )(dp3, x, w)


def _tc_post_body(dp_ref, p_ref, y_ref, b_ref, v_ref, st_ref):
    i = pl.program_id(0)
    dinv = _dinv_of(dp_ref[...])
    p = p_ref[...]
    v = dinv * (p[0] + p[1] + y_ref[...]) + b_ref[...][None, :]
    v_ref[...] = v

    @pl.when(i == 0)
    def _():
        st_ref[...] = jnp.zeros_like(st_ref)
    st_ref[0:1, :] += jnp.sum(v, axis=0, keepdims=True)
    st_ref[1:2, :] += jnp.sum(v * v, axis=0, keepdims=True)


def _tc_post(dp3, parts, y, b):
    n, fdim = y.shape
    grid = (n // _BLK,)
    return pl.pallas_call(
        _tc_post_body,
        grid=grid,
        in_specs=[
            pl.BlockSpec((_NC, _BLK, 1), lambda i: (0, i, 0)),
            pl.BlockSpec((_NC, _BLK, fdim), lambda i: (0, i, 0)),
            pl.BlockSpec((_BLK, fdim), lambda i: (i, 0)),
            pl.BlockSpec((fdim,), lambda i: (0,)),
        ],
        out_specs=[
            pl.BlockSpec((_BLK, fdim), lambda i: (i, 0)),
            pl.BlockSpec((8, fdim), lambda i: (0, 0)),
        ],
        out_shape=[
            jax.ShapeDtypeStruct((n, fdim), jnp.float32),
            jax.ShapeDtypeStruct((8, fdim), jnp.float32),
        ],
    )(dp3, parts, y, b)


def _bn_apply(v, st, g_row, b_row, n_nodes):
    mean = st[0:1, :] * (1.0 / n_nodes)
    var = st[1:2, :] * (1.0 / n_nodes) - mean * mean
    return g_row[None, :] * ((v - mean) * lax.rsqrt(var + 1e-5)) + b_row[None, :]


def _tc_mm2_body(dp_ref, v_ref, st_ref, g_ref, be_ref, w_ref, y2_ref, *, n_nodes):
    z = _bn_apply(v_ref[...], st_ref[...], g_ref[...], be_ref[...], n_nodes)
    h = jnp.where(z >= 0, z, 0.01 * z)
    dinv = _dinv_of(dp_ref[...])
    y2_ref[...] = dinv * jnp.dot(h, w_ref[...],
                                 preferred_element_type=jnp.float32)


def _tc_mm2(dp3, v, st, g, be, w):
    n, fdim = v.shape
    grid = (n // _BLK,)
    return pl.pallas_call(
        functools.partial(_tc_mm2_body, n_nodes=float(n)),
        grid=grid,
        in_specs=[
            pl.BlockSpec((_NC, _BLK, 1), lambda i: (0, i, 0)),
            pl.BlockSpec((_BLK, fdim), lambda i: (i, 0)),
            pl.BlockSpec((8, fdim), lambda i: (0, 0)),
            pl.BlockSpec((fdim,), lambda i: (0,)),
            pl.BlockSpec((fdim,), lambda i: (0,)),
            pl.BlockSpec((fdim, w.shape[1]), lambda i: (0, 0)),
        ],
        out_specs=pl.BlockSpec((_BLK, w.shape[1]), lambda i: (i, 0)),
        out_shape=jax.ShapeDtypeStruct((n, w.shape[1]), jnp.float32),
    )(dp3, v, st, g, be, w)


def _tc_final_body(v_ref, st_ref, g_ref, be_ref, o_ref, *, n_nodes):
    o_ref[...] = _bn_apply(v_ref[...], st_ref[...], g_ref[...], be_ref[...],
                           n_nodes)


def _tc_final(v, st, g, be):
    n, fdim = v.shape
    grid = (n // _BLK,)
    return pl.pallas_call(
        functools.partial(_tc_final_body, n_nodes=float(n)),
        grid=grid,
        in_specs=[
            pl.BlockSpec((_BLK, fdim), lambda i: (i, 0)),
            pl.BlockSpec((8, fdim), lambda i: (0, 0)),
            pl.BlockSpec((fdim,), lambda i: (0,)),
            pl.BlockSpec((fdim,), lambda i: (0,)),
        ],
        out_specs=pl.BlockSpec((_BLK, fdim), lambda i: (i, 0)),
        out_shape=jax.ShapeDtypeStruct((n, fdim), jnp.float32),
    )(v, st, g, be)


def kernel(ATC_adj, ATC_weight, drug_smiles_fea, W1, b1, gamma1, beta1,
           W2, b2, gamma2, beta2):
    x = drug_smiles_fea
    n, fdim = x.shape
    e_total = ATC_weight.shape[0]
    ntiles = _NC * _NS
    epw = e_total // ntiles
    epw_p = ((epw + fdim - 1) // fdim) * fdim

    def _plane(a, dtype):
        a = a.astype(dtype).reshape(ntiles, epw)
        a = jnp.pad(a, ((0, 0), (0, epw_p - epw)))
        return a.reshape(ntiles, epw_p // fdim, fdim)

    row3 = _plane(ATC_adj[0], jnp.int32)
    col3 = _plane(ATC_adj[1], jnp.int32)
    ew3 = _plane(ATC_weight, jnp.float32)
    zeros1 = jnp.zeros((n,), jnp.float32)
    zeros2 = jnp.zeros((n, fdim), jnp.float32)

    dparts = _sc_deg(col3, ew3, zeros1, n)
    dp3 = dparts.reshape(_NC, n, 1)

    y1 = _tc_mm1(dp3, x, W1)
    parts1 = _sc_agg(row3, col3, ew3, y1, zeros2)
    v1, st1 = _tc_post(dp3, parts1, y1, b1)

    y2 = _tc_mm2(dp3, v1, st1, gamma1, beta1, W2)
    parts2 = _sc_agg(row3, col3, ew3, y2, zeros2)
    v2, st2 = _tc_post(dp3, parts2, y2, b2)

    return _tc_final(v2, st2, gamma2, beta2)


# speedup vs baseline: 11.5993x; 1.1451x over previous
import functools

import jax
import jax.numpy as jnp
from jax import lax
from jax.experimental import pallas as pl
from jax.experimental.pallas import tpu as pltpu
from jax.experimental.pallas import tpu_sc as plsc

_NC = 2
_NS = 16
_L = 16
_C = 80
_BLK = 1000


def _sc_deg_body(col_hbm, ew_hbm, z_hbm, out_hbm, col_v, ew_v, acc_sh, sem):
    cid = lax.axis_index("c")
    sid = lax.axis_index("s")
    wid = cid * _NS + sid
    nch = col_v.shape[0]
    n = acc_sh.shape[0]

    pltpu.sync_copy(col_hbm.at[wid], col_v)
    pltpu.sync_copy(ew_hbm.at[wid], ew_v)

    @pl.when(sid == 0)
    def _():
        pltpu.sync_copy(z_hbm, acc_sh)
    plsc.subcore_barrier()

    def _fire(i, _):
        pltpu.async_copy(ew_v.at[i], acc_sh.at[col_v.at[i]], sem, add=True)
        return 0
    lax.fori_loop(0, nch, _fire, 0)

    def _drain(i, _):
        pltpu.make_async_copy(ew_v.at[0], acc_sh.at[col_v.at[0]], sem).wait()
        return 0
    lax.fori_loop(0, nch, _drain, 0)
    plsc.subcore_barrier()

    @pl.when(sid == 0)
    def _():
        pltpu.sync_copy(acc_sh, out_hbm.at[cid])


def _sc_deg(col3, ew3, zeros1, n):
    nch, c = col3.shape[1], col3.shape[2]
    mesh = plsc.VectorSubcoreMesh(core_axis_name="c", subcore_axis_name="s",
                                  num_cores=_NC, num_subcores=_NS)
    f = pl.kernel(
        _sc_deg_body,
        out_type=jax.ShapeDtypeStruct((_NC, n), jnp.float32),
        mesh=mesh,
        scratch_types=[
            pltpu.VMEM((nch, c), jnp.int32),
            pltpu.VMEM((nch, c), jnp.float32),
            pltpu.VMEM_SHARED((n,), jnp.float32),
            pltpu.SemaphoreType.DMA,
        ],
    )
    return f(col3, ew3, zeros1)


_BCH = 8


def _sc_agg_body(row_hbm, col_hbm, ew_hbm, y_hbm, z_hbm, out_hbm,
                 row0, row1, ew0, ew1, c0, c1, rows0, rows1,
                 acc_sh, gsem, ssem, isem, *, nch, c):
    cid = lax.axis_index("c")
    sid = lax.axis_index("s")
    wid = cid * _NS + sid
    n, fdim = y_hbm.shape
    nf = fdim // _L
    nwt = 10
    rpt = n // nwt
    rowv = (row0, row1)
    ewv = (ew0, ew1)
    colv = (c0, c1)
    rows = (rows0, rows1)
    base = wid * (nch * c)

    def _load_idx(i, b):
        off = base + i * c
        pltpu.async_copy(row_hbm.at[pl.ds(off, c)], rowv[b], isem)
        pltpu.async_copy(ew_hbm.at[pl.ds(off, c)], ewv[b], isem)
        pltpu.async_copy(col_hbm.at[pl.ds(off, c)], colv[b], isem)
        pltpu.make_async_copy(row_hbm.at[pl.ds(0, c)], rowv[b], isem).wait()
        pltpu.make_async_copy(ew_hbm.at[pl.ds(0, c)], ewv[b], isem).wait()
        pltpu.make_async_copy(col_hbm.at[pl.ds(0, c)], colv[b], isem).wait()

    @pl.when(sid < nwt)
    def _():
        pltpu.sync_copy(z_hbm.at[pl.ds(sid * rpt, rpt)],
                        acc_sh.at[pl.ds(sid * rpt, rpt)])
    plsc.subcore_barrier()

    def _outer(i, _):
        b = 0
        _load_idx(i, b)
        pltpu.async_copy(y_hbm.at[rowv[b]], rows[b], gsem).wait()

        rb = rows[b]
        ewb = ewv[b]
        if True:

            def _scale(k, _):
                ewl = ewb[pl.ds(k * _L, _L)]
                for j in range(_L):
                    s = lax.gather(
                        ewl, jnp.full((_L, 1), j, jnp.int32),
                        dimension_numbers=lax.GatherDimensionNumbers(
                            offset_dims=(), collapsed_slice_dims=(0,),
                            start_index_map=(0,)),
                        slice_sizes=(1,),
                        mode=lax.GatherScatterMode.PROMISE_IN_BOUNDS)
                    e = k * _L + j
                    for f in range(nf):
                        rb[e, pl.ds(f * _L, _L)] = (
                            rb[e, pl.ds(f * _L, _L)] * s)
                return 0
            lax.fori_loop(0, c // _L, _scale, 0)

            pltpu.async_copy(rows[b], acc_sh.at[colv[b]], ssem,
                             add=True).wait()
        return 0
    lax.fori_loop(0, nch, _outer, 0)
    plsc.subcore_barrier()

    @pl.when(sid < nwt)
    def _():
        pltpu.sync_copy(acc_sh.at[pl.ds(sid * rpt, rpt)],
                        out_hbm.at[cid, pl.ds(sid * rpt, rpt)])


def _sc_agg(row3, col3, ew3, y, zeros2):
    n, fdim = y.shape
    c = row3.shape[2]
    row1 = row3.reshape(-1)
    col1 = col3.reshape(-1)
    ew1 = ew3.reshape(-1)
    mesh = plsc.VectorSubcoreMesh(core_axis_name="c", subcore_axis_name="s",
                                  num_cores=_NC, num_subcores=_NS)
    f = pl.kernel(
        functools.partial(_sc_agg_body, nch=row3.shape[1], c=c),
        out_type=jax.ShapeDtypeStruct((_NC, n, fdim), jnp.float32),
        mesh=mesh,
        scratch_types=[
            pltpu.VMEM((c,), jnp.int32),
            pltpu.VMEM((c,), jnp.int32),
            pltpu.VMEM((c,), jnp.float32),
            pltpu.VMEM((c,), jnp.float32),
            pltpu.VMEM((c,), jnp.int32),
            pltpu.VMEM((c,), jnp.int32),
            pltpu.VMEM((c, fdim), jnp.float32),
            pltpu.VMEM((c, fdim), jnp.float32),
            pltpu.VMEM_SHARED((n, fdim), jnp.float32),
            pltpu.SemaphoreType.DMA,
            pltpu.SemaphoreType.DMA,
            pltpu.SemaphoreType.DMA,
        ],
    )
    return f(row1, col1, ew1, y, zeros2)


def _dinv_of(dp):
    return lax.rsqrt(1.0 + dp[0] + dp[1])


def _tc_mm1_body(dp_ref, x_ref, w_ref, y_ref):
    dinv = _dinv_of(dp_ref[...])
    y_ref[...] = dinv * jnp.dot(x_ref[...], w_ref[...],
                                preferred_element_type=jnp.float32)


def _tc_mm1(dp3, x, w):
    n, fdim = x.shape
    grid = (n // _BLK,)
    return pl.pallas_call(
        _tc_mm1_body,
        grid=grid,
        in_specs=[
            pl.BlockSpec((_NC, _BLK, 1), lambda i: (0, i, 0)),
            pl.BlockSpec((_BLK, fdim), lambda i: (i, 0)),
            pl.BlockSpec((fdim, w.shape[1]), lambda i: (0, 0)),
        ],
        out_specs=pl.BlockSpec((_BLK, w.shape[1]), lambda i: (i, 0)),
        out_shape=jax.ShapeDtypeStruct((n, w.shape[1]), jnp.float32),
    )(dp3, x, w)


def _tc_post_body(dp_ref, p_ref, y_ref, b_ref, v_ref, st_ref):
    i = pl.program_id(0)
    dinv = _dinv_of(dp_ref[...])
    p = p_ref[...]
    v = dinv * (p[0] + p[1] + y_ref[...]) + b_ref[...][None, :]
    v_ref[...] = v

    @pl.when(i == 0)
    def _():
        st_ref[...] = jnp.zeros_like(st_ref)
    st_ref[0:1, :] += jnp.sum(v, axis=0, keepdims=True)
    st_ref[1:2, :] += jnp.sum(v * v, axis=0, keepdims=True)


def _tc_post(dp3, parts, y, b):
    n, fdim = y.shape
    grid = (n // _BLK,)
    return pl.pallas_call(
        _tc_post_body,
        grid=grid,
        in_specs=[
            pl.BlockSpec((_NC, _BLK, 1), lambda i: (0, i, 0)),
            pl.BlockSpec((_NC, _BLK, fdim), lambda i: (0, i, 0)),
            pl.BlockSpec((_BLK, fdim), lambda i: (i, 0)),
            pl.BlockSpec((fdim,), lambda i: (0,)),
        ],
        out_specs=[
            pl.BlockSpec((_BLK, fdim), lambda i: (i, 0)),
            pl.BlockSpec((8, fdim), lambda i: (0, 0)),
        ],
        out_shape=[
            jax.ShapeDtypeStruct((n, fdim), jnp.float32),
            jax.ShapeDtypeStruct((8, fdim), jnp.float32),
        ],
    )(dp3, parts, y, b)


def _bn_apply(v, st, g_row, b_row, n_nodes):
    mean = st[0:1, :] * (1.0 / n_nodes)
    var = st[1:2, :] * (1.0 / n_nodes) - mean * mean
    return g_row[None, :] * ((v - mean) * lax.rsqrt(var + 1e-5)) + b_row[None, :]


def _tc_mm2_body(dp_ref, v_ref, st_ref, g_ref, be_ref, w_ref, y2_ref, *, n_nodes):
    z = _bn_apply(v_ref[...], st_ref[...], g_ref[...], be_ref[...], n_nodes)
    h = jnp.where(z >= 0, z, 0.01 * z)
    dinv = _dinv_of(dp_ref[...])
    y2_ref[...] = dinv * jnp.dot(h, w_ref[...],
                                 preferred_element_type=jnp.float32)


def _tc_mm2(dp3, v, st, g, be, w):
    n, fdim = v.shape
    grid = (n // _BLK,)
    return pl.pallas_call(
        functools.partial(_tc_mm2_body, n_nodes=float(n)),
        grid=grid,
        in_specs=[
            pl.BlockSpec((_NC, _BLK, 1), lambda i: (0, i, 0)),
            pl.BlockSpec((_BLK, fdim), lambda i: (i, 0)),
            pl.BlockSpec((8, fdim), lambda i: (0, 0)),
            pl.BlockSpec((fdim,), lambda i: (0,)),
            pl.BlockSpec((fdim,), lambda i: (0,)),
            pl.BlockSpec((fdim, w.shape[1]), lambda i: (0, 0)),
        ],
        out_specs=pl.BlockSpec((_BLK, w.shape[1]), lambda i: (i, 0)),
        out_shape=jax.ShapeDtypeStruct((n, w.shape[1]), jnp.float32),
    )(dp3, v, st, g, be, w)


def _tc_final_body(v_ref, st_ref, g_ref, be_ref, o_ref, *, n_nodes):
    o_ref[...] = _bn_apply(v_ref[...], st_ref[...], g_ref[...], be_ref[...],
                           n_nodes)


def _tc_final(v, st, g, be):
    n, fdim = v.shape
    grid = (n // _BLK,)
    return pl.pallas_call(
        functools.partial(_tc_final_body, n_nodes=float(n)),
        grid=grid,
        in_specs=[
            pl.BlockSpec((_BLK, fdim), lambda i: (i, 0)),
            pl.BlockSpec((8, fdim), lambda i: (0, 0)),
            pl.BlockSpec((fdim,), lambda i: (0,)),
            pl.BlockSpec((fdim,), lambda i: (0,)),
        ],
        out_specs=pl.BlockSpec((_BLK, fdim), lambda i: (i, 0)),
        out_shape=jax.ShapeDtypeStruct((n, fdim), jnp.float32),
    )(v, st, g, be)


def kernel(ATC_adj, ATC_weight, drug_smiles_fea, W1, b1, gamma1, beta1,
           W2, b2, gamma2, beta2):
    x = drug_smiles_fea
    n, fdim = x.shape
    e_total = ATC_weight.shape[0]
    ntiles = _NC * _NS
    epw = e_total // ntiles
    epw_p = ((epw + fdim - 1) // fdim) * fdim

    def _plane(a, dtype):
        a = a.astype(dtype).reshape(ntiles, epw)
        a = jnp.pad(a, ((0, 0), (0, epw_p - epw)))
        return a.reshape(ntiles, epw_p // fdim, fdim)

    row3 = _plane(ATC_adj[0], jnp.int32)
    col3 = _plane(ATC_adj[1], jnp.int32)
    ew3 = _plane(ATC_weight, jnp.float32)
    zeros1 = jnp.zeros((n,), jnp.float32)
    zeros2 = jnp.zeros((n, fdim), jnp.float32)

    dparts = _sc_deg(col3, ew3, zeros1, n)
    dp3 = dparts.reshape(_NC, n, 1)

    y1 = _tc_mm1(dp3, x, W1)
    parts1 = _sc_agg(row3, col3, ew3, y1, zeros2)
    v1, st1 = _tc_post(dp3, parts1, y1, b1)

    y2 = _tc_mm2(dp3, v1, st1, gamma1, beta1, W2)
    parts2 = _sc_agg(row3, col3, ew3, y2, zeros2)
    v2, st2 = _tc_post(dp3, parts2, y2, b2)

    return _tc_final(v2, st2, gamma2, beta2)
